# baseline (speedup 1.0000x reference)
import functools

import jax
import jax.numpy as jnp
from jax import lax
from jax.experimental import pallas as pl
from jax.experimental.pallas import tpu as pltpu
from jax.experimental.pallas import tpu_sc as plsc

N = 10000
E = 320000
D_IN = 128
D_HID = 64
D_OUT = 128
T = 1.0

NC = 2
NS = 16
NW = NC * NS
L = 16

NP = 10240
PAD = N
C = 128
E_TOT = E + N
CHUNKS = -(-E_TOT // (NW * C))
EP = NW * C * CHUNKS

_mesh = plsc.VectorSubcoreMesh(core_axis_name="c", subcore_axis_name="s")


def _signs_body(D, h_hbm, src_hbm, dst_hbm, pq_hbm, gidx_hbm,
                p_loc, q_loc, srcv0, srcv1, dstv0, dstv1, gidxv,
                srows0, srows1, drows0, drows1,
                sem10, sem11, sem20, sem21):
    c = lax.axis_index("c")
    s = lax.axis_index("s")
    w = c * NS + s

    srcvs = (srcv0, srcv1)
    dstvs = (dstv0, dstv1)
    srowss = (srows0, srows1)
    drowss = (drows0, drows1)
    sem1s = (sem10, sem11)
    sem2s = (sem20, sem21)

    zero16 = jnp.zeros((L,), jnp.float32)

    def zloop(i, carry):
        p_loc[pl.ds(i * L, L)] = zero16
        q_loc[pl.ds(i * L, L)] = zero16
        return carry
    lax.fori_loop(0, NP // L, zloop, 0)

    lane = lax.iota(jnp.int32, L)
    ones = jnp.ones((L,), jnp.float32)

    def issue(ci, b):
        e0 = (w * CHUNKS + ci) * C
        pltpu.sync_copy(src_hbm.at[pl.ds(e0, C)], srcvs[b])
        pltpu.sync_copy(dst_hbm.at[pl.ds(e0, C)], dstvs[b])
        pltpu.async_copy(h_hbm.at[srcvs[b]], srowss[b], sem1s[b])
        pltpu.async_copy(h_hbm.at[dstvs[b]], drowss[b], sem2s[b])

    def compute(ci, b):
        srcv = srcvs[b]
        srows = srowss[b]
        drows = drowss[b]
        e0 = (w * CHUNKS + ci) * C
        for g in range(C // L):
            eidx = lane + g * L

            def dblk(db, a):
                for dj in range(L):
                    dd = lane * 0 + (db * L + dj)
                    a = a + (plsc.load_gather(srows, [eidx, dd])
                             * plsc.load_gather(drows, [eidx, dd]))
                return a
            acc = lax.fori_loop(0, D // L, dblk, jnp.zeros((L,), jnp.float32))
            pos = acc > 0.0
            srcg = srcv[pl.ds(g * L, L)]
            plsc.addupdate_scatter(p_loc, [srcg], ones, mask=pos)
            plsc.addupdate_scatter(q_loc, [srcg], ones,
                                   mask=jnp.logical_not(pos))
            gidxv[pl.ds(g * L, L)] = srcg + jnp.where(
                pos, jnp.int32(NP), jnp.int32(0))
        pltpu.sync_copy(gidxv, gidx_hbm.at[pl.ds(e0, C)])

    issue(0, 0)

    def pair(pi, carry):
        for b in range(2):
            ci = pi * 2 + b

            @pl.when(ci < CHUNKS)
            def _():
                @pl.when(ci + 1 < CHUNKS)
                def _():
                    issue(ci + 1, b ^ 1)
                pltpu.make_async_copy(
                    h_hbm.at[srcvs[b]], srowss[b], sem1s[b]).wait()
                pltpu.make_async_copy(
                    h_hbm.at[dstvs[b]], drowss[b], sem2s[b]).wait()
                compute(ci, b)
        return carry
    lax.fori_loop(0, (CHUNKS + 1) // 2, pair, 0)

    pltpu.sync_copy(p_loc, pq_hbm.at[w, 0])
    pltpu.sync_copy(q_loc, pq_hbm.at[w, 1])


def _make_signs(D):
    return pl.kernel(
        functools.partial(_signs_body, D),
        out_type=(
            jax.ShapeDtypeStruct((NW, 2, NP), jnp.float32),
            jax.ShapeDtypeStruct((EP,), jnp.int32),
        ),
        mesh=_mesh,
        compiler_params=pltpu.CompilerParams(needs_layout_passes=False, use_tc_tiling_on_sc=False),
        scratch_types=[
            pltpu.VMEM((NP,), jnp.float32),
            pltpu.VMEM((NP,), jnp.float32),
            pltpu.VMEM((C,), jnp.int32),
            pltpu.VMEM((C,), jnp.int32),
            pltpu.VMEM((C,), jnp.int32),
            pltpu.VMEM((C,), jnp.int32),
            pltpu.VMEM((C,), jnp.int32),
            pltpu.VMEM((C, D), jnp.float32),
            pltpu.VMEM((C, D), jnp.float32),
            pltpu.VMEM((C, D), jnp.float32),
            pltpu.VMEM((C, D), jnp.float32),
            pltpu.SemaphoreType.DMA,
            pltpu.SemaphoreType.DMA,
            pltpu.SemaphoreType.DMA,
            pltpu.SemaphoreType.DMA,
        ],
    )


def _scatter_body(D, g_hbm, gidx_hbm, dst_hbm, out_hbm,
                  gidxv0, gidxv1, dstv0, dstv1, rows0, rows1,
                  zbuf, osh, sem0, sem1):
    c = lax.axis_index("c")
    s = lax.axis_index("s")
    w = c * NS + s
    stripe = NP // NS

    gidxvs = (gidxv0, gidxv1)
    dstvs = (dstv0, dstv1)
    rowss = (rows0, rows1)
    sems = (sem0, sem1)

    zero16 = jnp.zeros((L,), jnp.float32)

    def zrow(r, carry):
        for j in range(D // L):
            zbuf[r, pl.ds(j * L, L)] = zero16
        return carry
    lax.fori_loop(0, stripe, zrow, 0)

    pltpu.sync_copy(zbuf, osh.at[pl.ds(s * stripe, stripe)])
    plsc.subcore_barrier()

    def issue(ci, b):
        e0 = (w * CHUNKS + ci) * C
        pltpu.sync_copy(gidx_hbm.at[pl.ds(e0, C)], gidxvs[b])
        pltpu.sync_copy(dst_hbm.at[pl.ds(e0, C)], dstvs[b])
        pltpu.async_copy(g_hbm.at[gidxvs[b]], rowss[b], sems[b])

    issue(0, 0)

    def pair(pi, carry):
        for b in range(2):
            ci = pi * 2 + b

            @pl.when(ci < CHUNKS)
            def _():
                @pl.when(ci + 1 < CHUNKS)
                def _():
                    issue(ci + 1, b ^ 1)
                pltpu.make_async_copy(
                    g_hbm.at[gidxvs[b]], rowss[b], sems[b]).wait()
                pltpu.sync_copy(rowss[b], osh.at[dstvs[b]], add=True)
        return carry
    lax.fori_loop(0, (CHUNKS + 1) // 2, pair, 0)

    plsc.subcore_barrier()
    pltpu.sync_copy(osh.at[pl.ds(s * stripe, stripe)],
                    out_hbm.at[c, pl.ds(s * stripe, stripe)])


def _make_scatter(D):
    return pl.kernel(
        functools.partial(_scatter_body, D),
        out_type=jax.ShapeDtypeStruct((NC, NP, D), jnp.float32),
        mesh=_mesh,
        compiler_params=pltpu.CompilerParams(needs_layout_passes=False, use_tc_tiling_on_sc=False),
        scratch_types=[
            pltpu.VMEM((C,), jnp.int32),
            pltpu.VMEM((C,), jnp.int32),
            pltpu.VMEM((C,), jnp.int32),
            pltpu.VMEM((C,), jnp.int32),
            pltpu.VMEM((C, D), jnp.float32),
            pltpu.VMEM((C, D), jnp.float32),
            pltpu.VMEM((NP // NS, D), jnp.float32),
            pltpu.VMEM_SHARED((NP, D), jnp.float32),
            pltpu.SemaphoreType.DMA,
            pltpu.SemaphoreType.DMA,
        ],
    )


_BLK = 1024


def _mm_body(x_ref, w_ref, o_ref):
    o_ref[...] = lax.dot_general(
        x_ref[...], w_ref[...], (((1,), (1,)), ((), ())),
        preferred_element_type=jnp.float32)


def _matmul(x, w, d_out):
    npad, d_in = x.shape
    return pl.pallas_call(
        _mm_body,
        grid=(npad // _BLK,),
        in_specs=[
            pl.BlockSpec((_BLK, d_in), lambda i: (i, 0)),
            pl.BlockSpec((d_out, d_in), lambda i: (0, 0)),
        ],
        out_specs=pl.BlockSpec((_BLK, d_out), lambda i: (i, 0)),
        out_shape=jax.ShapeDtypeStruct((npad, d_out), jnp.float32),
    )(x, w)


def _gtable_body(h_ref, pq_ref, o_ref):
    g = pl.program_id(0)
    i = pl.program_id(1)
    p = jnp.sum(pq_ref[:, 0, pl.ds(i * _BLK, _BLK)], axis=0)
    q = jnp.sum(pq_ref[:, 1, pl.ds(i * _BLK, _BLK)], axis=0)
    amax = jnp.where(p > 0.0, T, -T)
    epos = jnp.exp(T - amax)
    eneg = jnp.exp(-T - amax)
    ssum = p * epos + q * eneg + 1e-16
    wsel = jnp.where(g == 0, eneg, epos) / ssum
    o_ref[...] = h_ref[...] * wsel[:, None]


def _gtable(h, pq, d):
    return pl.pallas_call(
        _gtable_body,
        grid=(2, NP // _BLK),
        in_specs=[
            pl.BlockSpec((_BLK, d), lambda g, i: (i, 0)),
            pl.BlockSpec((NW, 2, NP), lambda g, i: (0, 0, 0)),
        ],
        out_specs=pl.BlockSpec(
            (_BLK, d), lambda g, i: (g * (NP // _BLK) + i, 0)),
        out_shape=jax.ShapeDtypeStruct((2 * NP, d), jnp.float32),
    )(h, pq)


def _gtable_split_body(h_ref, pq_ref, o_ref):
    g = pl.program_id(0)
    hf = pl.program_id(1)
    i = pl.program_id(2)
    p = jnp.sum(pq_ref[:, 0, pl.ds(i * _BLK, _BLK)], axis=0)
    q = jnp.sum(pq_ref[:, 1, pl.ds(i * _BLK, _BLK)], axis=0)
    amax = jnp.where(p > 0.0, T, -T)
    epos = jnp.exp(T - amax)
    eneg = jnp.exp(-T - amax)
    ssum = p * epos + q * eneg + 1e-16
    wsel = jnp.where(g == 0, eneg, epos) / ssum
    half = D_OUT // 2
    hsel = jnp.where(hf == 0, h_ref[:, :half], h_ref[:, half:])
    o_ref[0] = hsel * wsel[:, None]


def _gtable_split(h, pq):
    return pl.pallas_call(
        _gtable_split_body,
        grid=(2, 2, NP // _BLK),
        in_specs=[
            pl.BlockSpec((_BLK, D_OUT), lambda g, hf, i: (i, 0)),
            pl.BlockSpec((NW, 2, NP), lambda g, hf, i: (0, 0, 0)),
        ],
        out_specs=pl.BlockSpec(
            (1, _BLK, D_OUT // 2),
            lambda g, hf, i: (hf, g * (NP // _BLK) + i, 0)),
        out_shape=jax.ShapeDtypeStruct((2, 2 * NP, D_OUT // 2), jnp.float32),
    )(h, pq)


def _combine_mm_body(op_ref, b_ref, w_ref, o_ref):
    i = pl.program_id(0)
    z = op_ref[0] + op_ref[1] + b_ref[...][None, :]
    z = jnp.maximum(z, 0.0)
    row = i * _BLK + lax.broadcasted_iota(jnp.int32, (_BLK, 1), 0)
    z = jnp.where(row < N, z, 0.0)
    o_ref[...] = lax.dot_general(
        z, w_ref[...], (((1,), (1,)), ((), ())),
        preferred_element_type=jnp.float32)


def _combine_mm(op, b, w, d_in, d_out):
    return pl.pallas_call(
        _combine_mm_body,
        grid=(NP // _BLK,),
        in_specs=[
            pl.BlockSpec((NC, _BLK, d_in), lambda i: (0, i, 0)),
            pl.BlockSpec((d_in,), lambda i: (0,)),
            pl.BlockSpec((d_out, d_in), lambda i: (0, 0)),
        ],
        out_specs=pl.BlockSpec((_BLK, d_out), lambda i: (i, 0)),
        out_shape=jax.ShapeDtypeStruct((NP, d_out), jnp.float32),
    )(op, b, w)


_OBLK = 400


def _final_body(opa_ref, opb_ref, b_ref, o_ref):
    z = jnp.concatenate(
        [opa_ref[0] + opa_ref[1], opb_ref[0] + opb_ref[1]], axis=1)
    z = z + b_ref[...][None, :]
    m = jnp.max(z, axis=1, keepdims=True)
    ez = jnp.exp(z - m)
    lse = jnp.log(jnp.sum(ez, axis=1, keepdims=True))
    o_ref[...] = z - m - lse


def _final(opa, opb, b):
    return pl.pallas_call(
        _final_body,
        grid=(N // _OBLK,),
        in_specs=[
            pl.BlockSpec((NC, _OBLK, D_OUT // 2), lambda i: (0, i, 0)),
            pl.BlockSpec((NC, _OBLK, D_OUT // 2), lambda i: (0, i, 0)),
            pl.BlockSpec((D_OUT,), lambda i: (0,)),
        ],
        out_specs=pl.BlockSpec((_OBLK, D_OUT), lambda i: (i, 0)),
        out_shape=jax.ShapeDtypeStruct((N, D_OUT), jnp.float32),
    )(opa, opb, b)


_signs64 = _make_signs(D_HID)
_signs128 = _make_signs(D_OUT)
_scatter64 = _make_scatter(D_HID)


def kernel(x, edge_index, W1, b1, W2, b2):
    loop = jnp.arange(N, dtype=jnp.int32)
    padv = jnp.full((EP - E_TOT,), PAD, dtype=jnp.int32)
    src = jnp.concatenate([edge_index[0], loop, padv])
    dst = jnp.concatenate([edge_index[1], loop, padv])

    xp = jnp.pad(x, ((0, NP - N), (0, 0)))

    h1 = _matmul(xp, W1, D_HID)
    pq1, gidx1 = _signs64(h1, src, dst)
    g1 = _gtable(h1, pq1, D_HID)
    op1 = _scatter64(g1, gidx1, dst)
    h2 = _combine_mm(op1, b1, W2, D_HID, D_OUT)
    pq2, gidx2 = _signs128(h2, src, dst)
    g2 = _gtable_split(h2, pq2)
    op2a = _scatter64(g2[0], gidx2, dst)
    op2b = _scatter64(g2[1], gidx2, dst)
    return _final(op2a, op2b, b2)


# speedup vs baseline: 2.0659x; 2.0659x over previous
import functools

import jax
import jax.numpy as jnp
from jax import lax
from jax.experimental import pallas as pl
from jax.experimental.pallas import tpu as pltpu
from jax.experimental.pallas import tpu_sc as plsc

N = 10000
E = 320000
D_IN = 128
D_HID = 64
D_OUT = 128
T = 1.0

NC = 2
NS = 16
NW = NC * NS
L = 16

NP = 10240
PAD = N
C = 128
E_TOT = E + N
CHUNKS = -(-E_TOT // (NW * C))
EP = NW * C * CHUNKS

_mesh = plsc.VectorSubcoreMesh(core_axis_name="c", subcore_axis_name="s")


def _signs_body(D, h_hbm, src_hbm, dst_hbm, pq_hbm, gidx_hbm,
                p_loc, q_loc, srcv0, srcv1, dstv0, dstv1, gidxv,
                srows0, srows1, drows0, drows1,
                sem10, sem11, sem20, sem21):
    c = lax.axis_index("c")
    s = lax.axis_index("s")
    w = c * NS + s

    srcvs = (srcv0, srcv1)
    dstvs = (dstv0, dstv1)
    srowss = (srows0, srows1)
    drowss = (drows0, drows1)
    sem1s = (sem10, sem11)
    sem2s = (sem20, sem21)

    zero16 = jnp.zeros((L,), jnp.float32)

    def zloop(i, carry):
        p_loc[pl.ds(i * L, L)] = zero16
        q_loc[pl.ds(i * L, L)] = zero16
        return carry
    lax.fori_loop(0, NP // L, zloop, 0)

    lane = lax.iota(jnp.int32, L)
    ones = jnp.ones((L,), jnp.float32)

    def issue(ci, b):
        e0 = (w * CHUNKS + ci) * C
        pltpu.sync_copy(src_hbm.at[pl.ds(e0, C)], srcvs[b])
        pltpu.sync_copy(dst_hbm.at[pl.ds(e0, C)], dstvs[b])
        pltpu.async_copy(h_hbm.at[srcvs[b]], srowss[b], sem1s[b])
        pltpu.async_copy(h_hbm.at[dstvs[b]], drowss[b], sem2s[b])

    def compute(ci, b):
        srcv = srcvs[b]
        srows = srowss[b]
        drows = drowss[b]
        e0 = (w * CHUNKS + ci) * C

        def group(g, carry):
            dots = jnp.zeros((L,), jnp.float32)
            for l in range(L):
                e = g * L + l
                acc = (srows[e, pl.ds(0, L)] * drows[e, pl.ds(0, L)])
                for j in range(1, D // L):
                    acc = acc + (srows[e, pl.ds(j * L, L)]
                                 * drows[e, pl.ds(j * L, L)])
                dots = jnp.where(lane == l, jnp.sum(acc), dots)
            pos = dots > 0.0
            srcg = srcv[pl.ds(g * L, L)]
            plsc.addupdate_scatter(p_loc, [srcg], ones, mask=pos)
            plsc.addupdate_scatter(q_loc, [srcg], ones,
                                   mask=jnp.logical_not(pos))
            gidxv[pl.ds(g * L, L)] = srcg + jnp.where(
                pos, jnp.int32(NP), jnp.int32(0))
            return carry
        lax.fori_loop(0, C // L, group, 0)
        pltpu.sync_copy(gidxv, gidx_hbm.at[pl.ds(e0, C)])

    issue(0, 0)

    def pair(pi, carry):
        for b in range(2):
            ci = pi * 2 + b

            @pl.when(ci < CHUNKS)
            def _():
                @pl.when(ci + 1 < CHUNKS)
                def _():
                    issue(ci + 1, b ^ 1)
                pltpu.make_async_copy(
                    h_hbm.at[srcvs[b]], srowss[b], sem1s[b]).wait()
                pltpu.make_async_copy(
                    h_hbm.at[dstvs[b]], drowss[b], sem2s[b]).wait()
                compute(ci, b)
        return carry
    lax.fori_loop(0, (CHUNKS + 1) // 2, pair, 0)

    pltpu.sync_copy(p_loc, pq_hbm.at[w, 0])
    pltpu.sync_copy(q_loc, pq_hbm.at[w, 1])


def _make_signs(D):
    return pl.kernel(
        functools.partial(_signs_body, D),
        out_type=(
            jax.ShapeDtypeStruct((NW, 2, NP), jnp.float32),
            jax.ShapeDtypeStruct((EP,), jnp.int32),
        ),
        mesh=_mesh,
        compiler_params=pltpu.CompilerParams(needs_layout_passes=False, use_tc_tiling_on_sc=False),
        scratch_types=[
            pltpu.VMEM((NP,), jnp.float32),
            pltpu.VMEM((NP,), jnp.float32),
            pltpu.VMEM((C,), jnp.int32),
            pltpu.VMEM((C,), jnp.int32),
            pltpu.VMEM((C,), jnp.int32),
            pltpu.VMEM((C,), jnp.int32),
            pltpu.VMEM((C,), jnp.int32),
            pltpu.VMEM((C, D), jnp.float32),
            pltpu.VMEM((C, D), jnp.float32),
            pltpu.VMEM((C, D), jnp.float32),
            pltpu.VMEM((C, D), jnp.float32),
            pltpu.SemaphoreType.DMA,
            pltpu.SemaphoreType.DMA,
            pltpu.SemaphoreType.DMA,
            pltpu.SemaphoreType.DMA,
        ],
    )


def _scatter_body(D, g_hbm, gidx_hbm, dst_hbm, out_hbm,
                  gidxv0, gidxv1, dstv0, dstv1, rows0, rows1,
                  zbuf, osh, sem0, sem1):
    c = lax.axis_index("c")
    s = lax.axis_index("s")
    w = c * NS + s
    stripe = NP // NS

    gidxvs = (gidxv0, gidxv1)
    dstvs = (dstv0, dstv1)
    rowss = (rows0, rows1)
    sems = (sem0, sem1)

    zero16 = jnp.zeros((L,), jnp.float32)

    def zrow(r, carry):
        for j in range(D // L):
            zbuf[r, pl.ds(j * L, L)] = zero16
        return carry
    lax.fori_loop(0, stripe, zrow, 0)

    pltpu.sync_copy(zbuf, osh.at[pl.ds(s * stripe, stripe)])
    plsc.subcore_barrier()

    def issue(ci, b):
        e0 = (w * CHUNKS + ci) * C
        pltpu.sync_copy(gidx_hbm.at[pl.ds(e0, C)], gidxvs[b])
        pltpu.sync_copy(dst_hbm.at[pl.ds(e0, C)], dstvs[b])
        pltpu.async_copy(g_hbm.at[gidxvs[b]], rowss[b], sems[b])

    issue(0, 0)

    def pair(pi, carry):
        for b in range(2):
            ci = pi * 2 + b

            @pl.when(ci < CHUNKS)
            def _():
                @pl.when(ci + 1 < CHUNKS)
                def _():
                    issue(ci + 1, b ^ 1)
                pltpu.make_async_copy(
                    g_hbm.at[gidxvs[b]], rowss[b], sems[b]).wait()
                pltpu.sync_copy(rowss[b], osh.at[dstvs[b]], add=True)
        return carry
    lax.fori_loop(0, (CHUNKS + 1) // 2, pair, 0)

    plsc.subcore_barrier()
    pltpu.sync_copy(osh.at[pl.ds(s * stripe, stripe)],
                    out_hbm.at[c, pl.ds(s * stripe, stripe)])


def _make_scatter(D):
    return pl.kernel(
        functools.partial(_scatter_body, D),
        out_type=jax.ShapeDtypeStruct((NC, NP, D), jnp.float32),
        mesh=_mesh,
        compiler_params=pltpu.CompilerParams(needs_layout_passes=False, use_tc_tiling_on_sc=False),
        scratch_types=[
            pltpu.VMEM((C,), jnp.int32),
            pltpu.VMEM((C,), jnp.int32),
            pltpu.VMEM((C,), jnp.int32),
            pltpu.VMEM((C,), jnp.int32),
            pltpu.VMEM((C, D), jnp.float32),
            pltpu.VMEM((C, D), jnp.float32),
            pltpu.VMEM((NP // NS, D), jnp.float32),
            pltpu.VMEM_SHARED((NP, D), jnp.float32),
            pltpu.SemaphoreType.DMA,
            pltpu.SemaphoreType.DMA,
        ],
    )


_BLK = 1024


def _mm_body(x_ref, w_ref, o_ref):
    o_ref[...] = lax.dot_general(
        x_ref[...], w_ref[...], (((1,), (1,)), ((), ())),
        preferred_element_type=jnp.float32)


def _matmul(x, w, d_out):
    npad, d_in = x.shape
    return pl.pallas_call(
        _mm_body,
        grid=(npad // _BLK,),
        in_specs=[
            pl.BlockSpec((_BLK, d_in), lambda i: (i, 0)),
            pl.BlockSpec((d_out, d_in), lambda i: (0, 0)),
        ],
        out_specs=pl.BlockSpec((_BLK, d_out), lambda i: (i, 0)),
        out_shape=jax.ShapeDtypeStruct((npad, d_out), jnp.float32),
    )(x, w)


def _gtable_body(h_ref, pq_ref, o_ref):
    g = pl.program_id(0)
    i = pl.program_id(1)
    p = jnp.sum(pq_ref[:, 0, pl.ds(i * _BLK, _BLK)], axis=0)
    q = jnp.sum(pq_ref[:, 1, pl.ds(i * _BLK, _BLK)], axis=0)
    amax = jnp.where(p > 0.0, T, -T)
    epos = jnp.exp(T - amax)
    eneg = jnp.exp(-T - amax)
    ssum = p * epos + q * eneg + 1e-16
    wsel = jnp.where(g == 0, eneg, epos) / ssum
    o_ref[...] = h_ref[...] * wsel[:, None]


def _gtable(h, pq, d):
    return pl.pallas_call(
        _gtable_body,
        grid=(2, NP // _BLK),
        in_specs=[
            pl.BlockSpec((_BLK, d), lambda g, i: (i, 0)),
            pl.BlockSpec((NW, 2, NP), lambda g, i: (0, 0, 0)),
        ],
        out_specs=pl.BlockSpec(
            (_BLK, d), lambda g, i: (g * (NP // _BLK) + i, 0)),
        out_shape=jax.ShapeDtypeStruct((2 * NP, d), jnp.float32),
    )(h, pq)


def _gtable_split_body(h_ref, pq_ref, o_ref):
    g = pl.program_id(0)
    hf = pl.program_id(1)
    i = pl.program_id(2)
    p = jnp.sum(pq_ref[:, 0, pl.ds(i * _BLK, _BLK)], axis=0)
    q = jnp.sum(pq_ref[:, 1, pl.ds(i * _BLK, _BLK)], axis=0)
    amax = jnp.where(p > 0.0, T, -T)
    epos = jnp.exp(T - amax)
    eneg = jnp.exp(-T - amax)
    ssum = p * epos + q * eneg + 1e-16
    wsel = jnp.where(g == 0, eneg, epos) / ssum
    half = D_OUT // 2
    hsel = jnp.where(hf == 0, h_ref[:, :half], h_ref[:, half:])
    o_ref[0] = hsel * wsel[:, None]


def _gtable_split(h, pq):
    return pl.pallas_call(
        _gtable_split_body,
        grid=(2, 2, NP // _BLK),
        in_specs=[
            pl.BlockSpec((_BLK, D_OUT), lambda g, hf, i: (i, 0)),
            pl.BlockSpec((NW, 2, NP), lambda g, hf, i: (0, 0, 0)),
        ],
        out_specs=pl.BlockSpec(
            (1, _BLK, D_OUT // 2),
            lambda g, hf, i: (hf, g * (NP // _BLK) + i, 0)),
        out_shape=jax.ShapeDtypeStruct((2, 2 * NP, D_OUT // 2), jnp.float32),
    )(h, pq)


def _combine_mm_body(op_ref, b_ref, w_ref, o_ref):
    i = pl.program_id(0)
    z = op_ref[0] + op_ref[1] + b_ref[...][None, :]
    z = jnp.maximum(z, 0.0)
    row = i * _BLK + lax.broadcasted_iota(jnp.int32, (_BLK, 1), 0)
    z = jnp.where(row < N, z, 0.0)
    o_ref[...] = lax.dot_general(
        z, w_ref[...], (((1,), (1,)), ((), ())),
        preferred_element_type=jnp.float32)


def _combine_mm(op, b, w, d_in, d_out):
    return pl.pallas_call(
        _combine_mm_body,
        grid=(NP // _BLK,),
        in_specs=[
            pl.BlockSpec((NC, _BLK, d_in), lambda i: (0, i, 0)),
            pl.BlockSpec((d_in,), lambda i: (0,)),
            pl.BlockSpec((d_out, d_in), lambda i: (0, 0)),
        ],
        out_specs=pl.BlockSpec((_BLK, d_out), lambda i: (i, 0)),
        out_shape=jax.ShapeDtypeStruct((NP, d_out), jnp.float32),
    )(op, b, w)


_OBLK = 400


def _final_body(opa_ref, opb_ref, b_ref, o_ref):
    z = jnp.concatenate(
        [opa_ref[0] + opa_ref[1], opb_ref[0] + opb_ref[1]], axis=1)
    z = z + b_ref[...][None, :]
    m = jnp.max(z, axis=1, keepdims=True)
    ez = jnp.exp(z - m)
    lse = jnp.log(jnp.sum(ez, axis=1, keepdims=True))
    o_ref[...] = z - m - lse


def _final(opa, opb, b):
    return pl.pallas_call(
        _final_body,
        grid=(N // _OBLK,),
        in_specs=[
            pl.BlockSpec((NC, _OBLK, D_OUT // 2), lambda i: (0, i, 0)),
            pl.BlockSpec((NC, _OBLK, D_OUT // 2), lambda i: (0, i, 0)),
            pl.BlockSpec((D_OUT,), lambda i: (0,)),
        ],
        out_specs=pl.BlockSpec((_OBLK, D_OUT), lambda i: (i, 0)),
        out_shape=jax.ShapeDtypeStruct((N, D_OUT), jnp.float32),
    )(opa, opb, b)


_signs64 = _make_signs(D_HID)
_signs128 = _make_signs(D_OUT)
_scatter64 = _make_scatter(D_HID)


def kernel(x, edge_index, W1, b1, W2, b2):
    loop = jnp.arange(N, dtype=jnp.int32)
    padv = jnp.full((EP - E_TOT,), PAD, dtype=jnp.int32)
    src = jnp.concatenate([edge_index[0], loop, padv])
    dst = jnp.concatenate([edge_index[1], loop, padv])

    xp = jnp.pad(x, ((0, NP - N), (0, 0)))

    h1 = _matmul(xp, W1, D_HID)
    pq1, gidx1 = _signs64(h1, src, dst)
    g1 = _gtable(h1, pq1, D_HID)
    op1 = _scatter64(g1, gidx1, dst)
    h2 = _combine_mm(op1, b1, W2, D_HID, D_OUT)
    pq2, gidx2 = _signs128(h2, src, dst)
    g2 = _gtable_split(h2, pq2)
    op2a = _scatter64(g2[0], gidx2, dst)
    op2b = _scatter64(g2[1], gidx2, dst)
    return _final(op2a, op2b, b2)


# speedup vs baseline: 2.4360x; 1.1792x over previous
import functools

import jax
import jax.numpy as jnp
from jax import lax
from jax.experimental import pallas as pl
from jax.experimental.pallas import tpu as pltpu
from jax.experimental.pallas import tpu_sc as plsc

N = 10000
E = 320000
D_IN = 128
D_HID = 64
D_OUT = 128
T = 1.0

NC = 2
NS = 16
NW = NC * NS
L = 16

NP = 10240
PAD = N
C = 128
E_TOT = E + N
CHUNKS = -(-E_TOT // (NW * C))
EP = NW * C * CHUNKS

_mesh = plsc.VectorSubcoreMesh(core_axis_name="c", subcore_axis_name="s")


CE = CHUNKS * C


def _signs_body(D, h_hbm, src_hbm, dst_hbm, pq_hbm, gidx_hbm,
                p_loc, q_loc, src_all, dst_all, gidx_all,
                srows0, srows1, drows0, drows1,
                sem10, sem11, sem20, sem21):
    c = lax.axis_index("c")
    s = lax.axis_index("s")
    w = c * NS + s

    srowss = (srows0, srows1)
    drowss = (drows0, drows1)
    sem1s = (sem10, sem11)
    sem2s = (sem20, sem21)

    zero16 = jnp.zeros((L,), jnp.float32)

    def zloop(i, carry):
        p_loc[pl.ds(i * L, L)] = zero16
        q_loc[pl.ds(i * L, L)] = zero16
        return carry
    lax.fori_loop(0, NP // L, zloop, 0)

    e_base = w * CE
    pltpu.sync_copy(src_hbm.at[pl.ds(e_base, CE)], src_all)
    pltpu.sync_copy(dst_hbm.at[pl.ds(e_base, CE)], dst_all)

    lane = lax.iota(jnp.int32, L)
    ones = jnp.ones((L,), jnp.float32)

    def issue(ci, b):
        pltpu.async_copy(
            h_hbm.at[src_all.at[pl.ds(ci * C, C)]], srowss[b], sem1s[b])
        pltpu.async_copy(
            h_hbm.at[dst_all.at[pl.ds(ci * C, C)]], drowss[b], sem2s[b])

    def compute(ci, b):
        srows = srowss[b]
        drows = drowss[b]

        def group(g, carry):
            dots = jnp.zeros((L,), jnp.float32)
            for l in range(L):
                e = g * L + l
                acc = (srows[e, pl.ds(0, L)] * drows[e, pl.ds(0, L)])
                for j in range(1, D // L):
                    acc = acc + (srows[e, pl.ds(j * L, L)]
                                 * drows[e, pl.ds(j * L, L)])
                dots = jnp.where(lane == l, jnp.sum(acc), dots)
            pos = dots > 0.0
            srcg = src_all[pl.ds(ci * C + g * L, L)]
            plsc.addupdate_scatter(p_loc, [srcg], ones, mask=pos)
            plsc.addupdate_scatter(q_loc, [srcg], ones,
                                   mask=jnp.logical_not(pos))
            gidx_all[pl.ds(ci * C + g * L, L)] = srcg + jnp.where(
                pos, jnp.int32(NP), jnp.int32(0))
            return carry
        lax.fori_loop(0, C // L, group, 0)

    issue(0, 0)

    def pair(pi, carry):
        for b in range(2):
            ci = pi * 2 + b

            @pl.when(ci < CHUNKS)
            def _():
                @pl.when(ci + 1 < CHUNKS)
                def _():
                    issue(ci + 1, b ^ 1)
                pltpu.make_async_copy(
                    h_hbm.at[src_all.at[pl.ds(0, C)]],
                    srowss[b], sem1s[b]).wait()
                pltpu.make_async_copy(
                    h_hbm.at[dst_all.at[pl.ds(0, C)]],
                    drowss[b], sem2s[b]).wait()
                compute(ci, b)
        return carry
    lax.fori_loop(0, (CHUNKS + 1) // 2, pair, 0)

    pltpu.sync_copy(gidx_all, gidx_hbm.at[pl.ds(e_base, CE)])
    pltpu.sync_copy(p_loc, pq_hbm.at[w, 0])
    pltpu.sync_copy(q_loc, pq_hbm.at[w, 1])


def _make_signs(D):
    return pl.kernel(
        functools.partial(_signs_body, D),
        out_type=(
            jax.ShapeDtypeStruct((NW, 2, NP), jnp.float32),
            jax.ShapeDtypeStruct((EP,), jnp.int32),
        ),
        mesh=_mesh,
        compiler_params=pltpu.CompilerParams(needs_layout_passes=False, use_tc_tiling_on_sc=False),
        scratch_types=[
            pltpu.VMEM((NP,), jnp.float32),
            pltpu.VMEM((NP,), jnp.float32),
            pltpu.VMEM((CE,), jnp.int32),
            pltpu.VMEM((CE,), jnp.int32),
            pltpu.VMEM((CE,), jnp.int32),
            pltpu.VMEM((C, D), jnp.float32),
            pltpu.VMEM((C, D), jnp.float32),
            pltpu.VMEM((C, D), jnp.float32),
            pltpu.VMEM((C, D), jnp.float32),
            pltpu.SemaphoreType.DMA,
            pltpu.SemaphoreType.DMA,
            pltpu.SemaphoreType.DMA,
            pltpu.SemaphoreType.DMA,
        ],
    )


def _scatter_body(D, g_hbm, gidx2_hbm, dst2_hbm, out_hbm,
                  gidx_all, dst_all, rows0, rows1,
                  zbuf, osh, sem0, sem1):
    c = lax.axis_index("c")
    s = lax.axis_index("s")
    w = c * NS + s
    stripe = NP // NS

    rowss = (rows0, rows1)
    sems = (sem0, sem1)

    zero16 = jnp.zeros((L,), jnp.float32)

    def zrow(r, carry):
        for j in range(D // L):
            zbuf[r, pl.ds(j * L, L)] = zero16
        return carry
    lax.fori_loop(0, stripe, zrow, 0)

    pltpu.sync_copy(zbuf, osh.at[pl.ds(s * stripe, stripe)])

    pltpu.sync_copy(gidx2_hbm.at[pl.ds(w * CHUNKS, CHUNKS)], gidx_all)
    pltpu.sync_copy(dst2_hbm.at[pl.ds(w * CHUNKS, CHUNKS)], dst_all)
    plsc.subcore_barrier()

    def issue(ci, b):
        pltpu.async_copy(g_hbm.at[gidx_all.at[ci]], rowss[b], sems[b])

    issue(0, 0)

    def pair(pi, carry):
        for b in range(2):
            ci = pi * 2 + b

            @pl.when(ci < CHUNKS)
            def _():
                @pl.when(ci + 1 < CHUNKS)
                def _():
                    issue(ci + 1, b ^ 1)
                pltpu.make_async_copy(
                    g_hbm.at[gidx_all.at[0]], rowss[b], sems[b]).wait()
                pltpu.sync_copy(rowss[b], osh.at[dst_all.at[ci]], add=True)
        return carry
    lax.fori_loop(0, (CHUNKS + 1) // 2, pair, 0)

    plsc.subcore_barrier()
    pltpu.sync_copy(osh.at[pl.ds(s * stripe, stripe)],
                    out_hbm.at[c, pl.ds(s * stripe, stripe)])


def _make_scatter(D):
    return pl.kernel(
        functools.partial(_scatter_body, D),
        out_type=jax.ShapeDtypeStruct((NC, NP, D), jnp.float32),
        mesh=_mesh,
        compiler_params=pltpu.CompilerParams(needs_layout_passes=False, use_tc_tiling_on_sc=False),
        scratch_types=[
            pltpu.VMEM((CHUNKS, C), jnp.int32),
            pltpu.VMEM((CHUNKS, C), jnp.int32),
            pltpu.VMEM((C, D), jnp.float32),
            pltpu.VMEM((C, D), jnp.float32),
            pltpu.VMEM((NP // NS, D), jnp.float32),
            pltpu.VMEM_SHARED((NP, D), jnp.float32),
            pltpu.SemaphoreType.DMA,
            pltpu.SemaphoreType.DMA,
        ],
    )


_BLK = 1024


def _mm_body(x_ref, w_ref, o_ref):
    o_ref[...] = lax.dot_general(
        x_ref[...], w_ref[...], (((1,), (1,)), ((), ())),
        preferred_element_type=jnp.float32)


def _matmul(x, w, d_out):
    npad, d_in = x.shape
    return pl.pallas_call(
        _mm_body,
        grid=(npad // _BLK,),
        in_specs=[
            pl.BlockSpec((_BLK, d_in), lambda i: (i, 0)),
            pl.BlockSpec((d_out, d_in), lambda i: (0, 0)),
        ],
        out_specs=pl.BlockSpec((_BLK, d_out), lambda i: (i, 0)),
        out_shape=jax.ShapeDtypeStruct((npad, d_out), jnp.float32),
    )(x, w)


def _gtable_body(h_ref, pq_ref, o_ref):
    g = pl.program_id(0)
    i = pl.program_id(1)
    p = jnp.sum(pq_ref[:, 0, pl.ds(i * _BLK, _BLK)], axis=0)
    q = jnp.sum(pq_ref[:, 1, pl.ds(i * _BLK, _BLK)], axis=0)
    amax = jnp.where(p > 0.0, T, -T)
    epos = jnp.exp(T - amax)
    eneg = jnp.exp(-T - amax)
    ssum = p * epos + q * eneg + 1e-16
    wsel = jnp.where(g == 0, eneg, epos) / ssum
    o_ref[...] = h_ref[...] * wsel[:, None]


def _gtable(h, pq, d):
    return pl.pallas_call(
        _gtable_body,
        grid=(2, NP // _BLK),
        in_specs=[
            pl.BlockSpec((_BLK, d), lambda g, i: (i, 0)),
            pl.BlockSpec((NW, 2, NP), lambda g, i: (0, 0, 0)),
        ],
        out_specs=pl.BlockSpec(
            (_BLK, d), lambda g, i: (g * (NP // _BLK) + i, 0)),
        out_shape=jax.ShapeDtypeStruct((2 * NP, d), jnp.float32),
    )(h, pq)


def _gtable_split_body(h_ref, pq_ref, o_ref):
    g = pl.program_id(0)
    hf = pl.program_id(1)
    i = pl.program_id(2)
    p = jnp.sum(pq_ref[:, 0, pl.ds(i * _BLK, _BLK)], axis=0)
    q = jnp.sum(pq_ref[:, 1, pl.ds(i * _BLK, _BLK)], axis=0)
    amax = jnp.where(p > 0.0, T, -T)
    epos = jnp.exp(T - amax)
    eneg = jnp.exp(-T - amax)
    ssum = p * epos + q * eneg + 1e-16
    wsel = jnp.where(g == 0, eneg, epos) / ssum
    half = D_OUT // 2
    hsel = jnp.where(hf == 0, h_ref[:, :half], h_ref[:, half:])
    o_ref[0] = hsel * wsel[:, None]


def _gtable_split(h, pq):
    return pl.pallas_call(
        _gtable_split_body,
        grid=(2, 2, NP // _BLK),
        in_specs=[
            pl.BlockSpec((_BLK, D_OUT), lambda g, hf, i: (i, 0)),
            pl.BlockSpec((NW, 2, NP), lambda g, hf, i: (0, 0, 0)),
        ],
        out_specs=pl.BlockSpec(
            (1, _BLK, D_OUT // 2),
            lambda g, hf, i: (hf, g * (NP // _BLK) + i, 0)),
        out_shape=jax.ShapeDtypeStruct((2, 2 * NP, D_OUT // 2), jnp.float32),
    )(h, pq)


def _combine_mm_body(op_ref, b_ref, w_ref, o_ref):
    i = pl.program_id(0)
    z = op_ref[0] + op_ref[1] + b_ref[...][None, :]
    z = jnp.maximum(z, 0.0)
    row = i * _BLK + lax.broadcasted_iota(jnp.int32, (_BLK, 1), 0)
    z = jnp.where(row < N, z, 0.0)
    o_ref[...] = lax.dot_general(
        z, w_ref[...], (((1,), (1,)), ((), ())),
        preferred_element_type=jnp.float32)


def _combine_mm(op, b, w, d_in, d_out):
    return pl.pallas_call(
        _combine_mm_body,
        grid=(NP // _BLK,),
        in_specs=[
            pl.BlockSpec((NC, _BLK, d_in), lambda i: (0, i, 0)),
            pl.BlockSpec((d_in,), lambda i: (0,)),
            pl.BlockSpec((d_out, d_in), lambda i: (0, 0)),
        ],
        out_specs=pl.BlockSpec((_BLK, d_out), lambda i: (i, 0)),
        out_shape=jax.ShapeDtypeStruct((NP, d_out), jnp.float32),
    )(op, b, w)


_OBLK = 400


def _final_body(opa_ref, opb_ref, b_ref, o_ref):
    z = jnp.concatenate(
        [opa_ref[0] + opa_ref[1], opb_ref[0] + opb_ref[1]], axis=1)
    z = z + b_ref[...][None, :]
    m = jnp.max(z, axis=1, keepdims=True)
    ez = jnp.exp(z - m)
    lse = jnp.log(jnp.sum(ez, axis=1, keepdims=True))
    o_ref[...] = z - m - lse


def _final(opa, opb, b):
    return pl.pallas_call(
        _final_body,
        grid=(N // _OBLK,),
        in_specs=[
            pl.BlockSpec((NC, _OBLK, D_OUT // 2), lambda i: (0, i, 0)),
            pl.BlockSpec((NC, _OBLK, D_OUT // 2), lambda i: (0, i, 0)),
            pl.BlockSpec((D_OUT,), lambda i: (0,)),
        ],
        out_specs=pl.BlockSpec((_OBLK, D_OUT), lambda i: (i, 0)),
        out_shape=jax.ShapeDtypeStruct((N, D_OUT), jnp.float32),
    )(opa, opb, b)


_signs64 = _make_signs(D_HID)
_signs128 = _make_signs(D_OUT)
_scatter64 = _make_scatter(D_HID)


def kernel(x, edge_index, W1, b1, W2, b2):
    loop = jnp.arange(N, dtype=jnp.int32)
    padv = jnp.full((EP - E_TOT,), PAD, dtype=jnp.int32)
    src = jnp.concatenate([edge_index[0], loop, padv])
    dst = jnp.concatenate([edge_index[1], loop, padv])

    xp = jnp.pad(x, ((0, NP - N), (0, 0)))

    dst2 = dst.reshape(-1, C)

    h1 = _matmul(xp, W1, D_HID)
    pq1, gidx1 = _signs64(h1, src, dst)
    g1 = _gtable(h1, pq1, D_HID)
    op1 = _scatter64(g1, gidx1.reshape(-1, C), dst2)
    h2 = _combine_mm(op1, b1, W2, D_HID, D_OUT)
    pq2, gidx2 = _signs128(h2, src, dst)
    g2 = _gtable_split(h2, pq2)
    gidx2r = gidx2.reshape(-1, C)
    op2a = _scatter64(g2[0], gidx2r, dst2)
    op2b = _scatter64(g2[1], gidx2r, dst2)
    return _final(op2a, op2b, b2)


# speedup vs baseline: 2.6593x; 1.0917x over previous
import functools

import jax
import jax.numpy as jnp
from jax import lax
from jax.experimental import pallas as pl
from jax.experimental.pallas import tpu as pltpu
from jax.experimental.pallas import tpu_sc as plsc

N = 10000
E = 320000
D_IN = 128
D_HID = 64
D_OUT = 128
T = 1.0

NC = 2
NS = 16
NW = NC * NS
L = 16

NP = 10240
PAD = N
C = 128
E_TOT = E + N
CHUNKS = -(-E_TOT // (NW * C))
EP = NW * C * CHUNKS
CH0 = 101
CH1 = 2 * CHUNKS - CH0
CHM = max(CH0, CH1)
CMIN = min(CH0, CH1)
EP_AL = EP + (CHM - CMIN) * C

_mesh = plsc.VectorSubcoreMesh(core_axis_name="c", subcore_axis_name="s")


CE = CHM * C


def _signs_body(D, h_hbm, src_hbm, dst_hbm, pq_hbm, gidx_hbm,
                p_loc, q_loc, src_all, dst_all, gidx_all,
                srows0, srows1, drows0, drows1,
                sem10, sem11, sem20, sem21):
    c = lax.axis_index("c")
    s = lax.axis_index("s")
    w = c * NS + s
    my_ch = jnp.where(c == 0, CH0, CH1)
    base_chunk = jnp.where(c == 0, s * CH0, NS * CH0 + s * CH1)

    srowss = (srows0, srows1)
    drowss = (drows0, drows1)
    sem1s = (sem10, sem11)
    sem2s = (sem20, sem21)

    zero16 = jnp.zeros((L,), jnp.float32)

    def zloop(i, carry):
        p_loc[pl.ds(i * L, L)] = zero16
        q_loc[pl.ds(i * L, L)] = zero16
        return carry
    lax.fori_loop(0, NP // L, zloop, 0)

    e_base = base_chunk * C
    pltpu.sync_copy(src_hbm.at[pl.ds(e_base, CE)], src_all)
    pltpu.sync_copy(dst_hbm.at[pl.ds(e_base, CE)], dst_all)

    lane = lax.iota(jnp.int32, L)
    ones = jnp.ones((L,), jnp.float32)

    def issue(ci, b):
        pltpu.async_copy(
            h_hbm.at[src_all.at[pl.ds(ci * C, C)]], srowss[b], sem1s[b])
        pltpu.async_copy(
            h_hbm.at[dst_all.at[pl.ds(ci * C, C)]], drowss[b], sem2s[b])

    def compute(ci, b):
        srows = srowss[b]
        drows = drowss[b]

        def group(g, carry):
            dots = jnp.zeros((L,), jnp.float32)
            for l in range(L):
                e = g * L + l
                acc = (srows[e, pl.ds(0, L)] * drows[e, pl.ds(0, L)])
                for j in range(1, D // L):
                    acc = acc + (srows[e, pl.ds(j * L, L)]
                                 * drows[e, pl.ds(j * L, L)])
                dots = jnp.where(lane == l, jnp.sum(acc), dots)
            pos = dots > 0.0
            srcg = src_all[pl.ds(ci * C + g * L, L)]
            plsc.addupdate_scatter(p_loc, [srcg], ones, mask=pos)
            plsc.addupdate_scatter(q_loc, [srcg], ones,
                                   mask=jnp.logical_not(pos))
            gidx_all[pl.ds(ci * C + g * L, L)] = srcg + jnp.where(
                pos, jnp.int32(NP), jnp.int32(0))
            return carry
        lax.fori_loop(0, C // L, group, 0)

    issue(0, 0)

    def pair(pi, carry):
        for b in range(2):
            ci = pi * 2 + b

            @pl.when(ci < my_ch)
            def _():
                @pl.when(ci + 1 < my_ch)
                def _():
                    issue(ci + 1, b ^ 1)
                pltpu.make_async_copy(
                    h_hbm.at[src_all.at[pl.ds(0, C)]],
                    srowss[b], sem1s[b]).wait()
                pltpu.make_async_copy(
                    h_hbm.at[dst_all.at[pl.ds(0, C)]],
                    drowss[b], sem2s[b]).wait()
                compute(ci, b)
        return carry
    lax.fori_loop(0, (my_ch + 1) // 2, pair, 0)

    pltpu.sync_copy(gidx_all.at[pl.ds(0, CMIN * C)],
                    gidx_hbm.at[pl.ds(e_base, CMIN * C)])

    @pl.when(my_ch > CMIN)
    def _():
        pltpu.sync_copy(
            gidx_all.at[pl.ds(CMIN * C, (CHM - CMIN) * C)],
            gidx_hbm.at[pl.ds(e_base + CMIN * C, (CHM - CMIN) * C)])
    pltpu.sync_copy(p_loc, pq_hbm.at[w, 0])
    pltpu.sync_copy(q_loc, pq_hbm.at[w, 1])


def _make_signs(D):
    return pl.kernel(
        functools.partial(_signs_body, D),
        out_type=(
            jax.ShapeDtypeStruct((NW, 2, NP), jnp.float32),
            jax.ShapeDtypeStruct((EP_AL,), jnp.int32),
        ),
        mesh=_mesh,
        compiler_params=pltpu.CompilerParams(needs_layout_passes=False, use_tc_tiling_on_sc=False),
        scratch_types=[
            pltpu.VMEM((NP,), jnp.float32),
            pltpu.VMEM((NP,), jnp.float32),
            pltpu.VMEM((CE,), jnp.int32),
            pltpu.VMEM((CE,), jnp.int32),
            pltpu.VMEM((CE,), jnp.int32),
            pltpu.VMEM((C, D), jnp.float32),
            pltpu.VMEM((C, D), jnp.float32),
            pltpu.VMEM((C, D), jnp.float32),
            pltpu.VMEM((C, D), jnp.float32),
            pltpu.SemaphoreType.DMA,
            pltpu.SemaphoreType.DMA,
            pltpu.SemaphoreType.DMA,
            pltpu.SemaphoreType.DMA,
        ],
    )


def _scatter_body(D, g_hbm, gidx2_hbm, dst2_hbm, out_hbm,
                  gidx_all, dst_all, rows0, rows1,
                  zbuf, osh, sem0, sem1):
    c = lax.axis_index("c")
    s = lax.axis_index("s")
    stripe = NP // NS
    my_ch = jnp.where(c == 0, CH0, CH1)
    base_chunk = jnp.where(c == 0, s * CH0, NS * CH0 + s * CH1)

    rowss = (rows0, rows1)
    sems = (sem0, sem1)

    zero16 = jnp.zeros((L,), jnp.float32)

    def zrow(r, carry):
        for j in range(D // L):
            zbuf[r, pl.ds(j * L, L)] = zero16
        return carry
    lax.fori_loop(0, stripe, zrow, 0)

    pltpu.sync_copy(zbuf, osh.at[pl.ds(s * stripe, stripe)])

    pltpu.sync_copy(gidx2_hbm.at[pl.ds(base_chunk, CHM)], gidx_all)
    pltpu.sync_copy(dst2_hbm.at[pl.ds(base_chunk, CHM)], dst_all)
    plsc.subcore_barrier()

    def issue(ci, b):
        pltpu.async_copy(g_hbm.at[gidx_all.at[ci]], rowss[b], sems[b])

    issue(0, 0)

    def pair(pi, carry):
        for b in range(2):
            ci = pi * 2 + b

            @pl.when(ci < my_ch)
            def _():
                @pl.when(ci + 1 < my_ch)
                def _():
                    issue(ci + 1, b ^ 1)
                pltpu.make_async_copy(
                    g_hbm.at[gidx_all.at[0]], rowss[b], sems[b]).wait()
                pltpu.sync_copy(rowss[b], osh.at[dst_all.at[ci]], add=True)
        return carry
    lax.fori_loop(0, (my_ch + 1) // 2, pair, 0)

    plsc.subcore_barrier()
    pltpu.sync_copy(osh.at[pl.ds(s * stripe, stripe)],
                    out_hbm.at[c, pl.ds(s * stripe, stripe)])


def _make_scatter(D):
    return pl.kernel(
        functools.partial(_scatter_body, D),
        out_type=jax.ShapeDtypeStruct((NC, NP, D), jnp.float32),
        mesh=_mesh,
        compiler_params=pltpu.CompilerParams(needs_layout_passes=False, use_tc_tiling_on_sc=False),
        scratch_types=[
            pltpu.VMEM((CHM, C), jnp.int32),
            pltpu.VMEM((CHM, C), jnp.int32),
            pltpu.VMEM((C, D), jnp.float32),
            pltpu.VMEM((C, D), jnp.float32),
            pltpu.VMEM((NP // NS, D), jnp.float32),
            pltpu.VMEM_SHARED((NP, D), jnp.float32),
            pltpu.SemaphoreType.DMA,
            pltpu.SemaphoreType.DMA,
        ],
    )


_BLK = 1024


def _mm_body(x_ref, w_ref, o_ref):
    o_ref[...] = lax.dot_general(
        x_ref[...], w_ref[...], (((1,), (1,)), ((), ())),
        preferred_element_type=jnp.float32)


def _matmul(x, w, d_out):
    npad, d_in = x.shape
    return pl.pallas_call(
        _mm_body,
        grid=(npad // _BLK,),
        in_specs=[
            pl.BlockSpec((_BLK, d_in), lambda i: (i, 0)),
            pl.BlockSpec((d_out, d_in), lambda i: (0, 0)),
        ],
        out_specs=pl.BlockSpec((_BLK, d_out), lambda i: (i, 0)),
        out_shape=jax.ShapeDtypeStruct((npad, d_out), jnp.float32),
    )(x, w)


def _gtable_body(h_ref, pq_ref, o_ref):
    g = pl.program_id(0)
    i = pl.program_id(1)
    p = jnp.sum(pq_ref[:, 0, pl.ds(i * _BLK, _BLK)], axis=0)
    q = jnp.sum(pq_ref[:, 1, pl.ds(i * _BLK, _BLK)], axis=0)
    amax = jnp.where(p > 0.0, T, -T)
    epos = jnp.exp(T - amax)
    eneg = jnp.exp(-T - amax)
    ssum = p * epos + q * eneg + 1e-16
    wsel = jnp.where(g == 0, eneg, epos) / ssum
    o_ref[...] = h_ref[...] * wsel[:, None]


def _gtable(h, pq, d):
    return pl.pallas_call(
        _gtable_body,
        grid=(2, NP // _BLK),
        in_specs=[
            pl.BlockSpec((_BLK, d), lambda g, i: (i, 0)),
            pl.BlockSpec((NW, 2, NP), lambda g, i: (0, 0, 0)),
        ],
        out_specs=pl.BlockSpec(
            (_BLK, d), lambda g, i: (g * (NP // _BLK) + i, 0)),
        out_shape=jax.ShapeDtypeStruct((2 * NP, d), jnp.float32),
    )(h, pq)


def _gtable_split_body(h_ref, pq_ref, o_ref):
    g = pl.program_id(0)
    hf = pl.program_id(1)
    i = pl.program_id(2)
    p = jnp.sum(pq_ref[:, 0, pl.ds(i * _BLK, _BLK)], axis=0)
    q = jnp.sum(pq_ref[:, 1, pl.ds(i * _BLK, _BLK)], axis=0)
    amax = jnp.where(p > 0.0, T, -T)
    epos = jnp.exp(T - amax)
    eneg = jnp.exp(-T - amax)
    ssum = p * epos + q * eneg + 1e-16
    wsel = jnp.where(g == 0, eneg, epos) / ssum
    half = D_OUT // 2
    hsel = jnp.where(hf == 0, h_ref[:, :half], h_ref[:, half:])
    o_ref[0] = hsel * wsel[:, None]


def _gtable_split(h, pq):
    return pl.pallas_call(
        _gtable_split_body,
        grid=(2, 2, NP // _BLK),
        in_specs=[
            pl.BlockSpec((_BLK, D_OUT), lambda g, hf, i: (i, 0)),
            pl.BlockSpec((NW, 2, NP), lambda g, hf, i: (0, 0, 0)),
        ],
        out_specs=pl.BlockSpec(
            (1, _BLK, D_OUT // 2),
            lambda g, hf, i: (hf, g * (NP // _BLK) + i, 0)),
        out_shape=jax.ShapeDtypeStruct((2, 2 * NP, D_OUT // 2), jnp.float32),
    )(h, pq)


def _combine_mm_body(op_ref, b_ref, w_ref, o_ref):
    i = pl.program_id(0)
    z = op_ref[0] + op_ref[1] + b_ref[...][None, :]
    z = jnp.maximum(z, 0.0)
    row = i * _BLK + lax.broadcasted_iota(jnp.int32, (_BLK, 1), 0)
    z = jnp.where(row < N, z, 0.0)
    o_ref[...] = lax.dot_general(
        z, w_ref[...], (((1,), (1,)), ((), ())),
        preferred_element_type=jnp.float32)


def _combine_mm(op, b, w, d_in, d_out):
    return pl.pallas_call(
        _combine_mm_body,
        grid=(NP // _BLK,),
        in_specs=[
            pl.BlockSpec((NC, _BLK, d_in), lambda i: (0, i, 0)),
            pl.BlockSpec((d_in,), lambda i: (0,)),
            pl.BlockSpec((d_out, d_in), lambda i: (0, 0)),
        ],
        out_specs=pl.BlockSpec((_BLK, d_out), lambda i: (i, 0)),
        out_shape=jax.ShapeDtypeStruct((NP, d_out), jnp.float32),
    )(op, b, w)


_OBLK = 400


def _final_body(opa_ref, opb_ref, b_ref, o_ref):
    z = jnp.concatenate(
        [opa_ref[0] + opa_ref[1], opb_ref[0] + opb_ref[1]], axis=1)
    z = z + b_ref[...][None, :]
    m = jnp.max(z, axis=1, keepdims=True)
    ez = jnp.exp(z - m)
    lse = jnp.log(jnp.sum(ez, axis=1, keepdims=True))
    o_ref[...] = z - m - lse


def _final(opa, opb, b):
    return pl.pallas_call(
        _final_body,
        grid=(N // _OBLK,),
        in_specs=[
            pl.BlockSpec((NC, _OBLK, D_OUT // 2), lambda i: (0, i, 0)),
            pl.BlockSpec((NC, _OBLK, D_OUT // 2), lambda i: (0, i, 0)),
            pl.BlockSpec((D_OUT,), lambda i: (0,)),
        ],
        out_specs=pl.BlockSpec((_OBLK, D_OUT), lambda i: (i, 0)),
        out_shape=jax.ShapeDtypeStruct((N, D_OUT), jnp.float32),
    )(opa, opb, b)


_signs64 = _make_signs(D_HID)
_signs128 = _make_signs(D_OUT)
_scatter64 = _make_scatter(D_HID)


def kernel(x, edge_index, W1, b1, W2, b2):
    loop = jnp.arange(N, dtype=jnp.int32)
    padv = jnp.full((EP_AL - E_TOT,), PAD, dtype=jnp.int32)
    src = jnp.concatenate([edge_index[0], loop, padv])
    dst = jnp.concatenate([edge_index[1], loop, padv])

    xp = jnp.pad(x, ((0, NP - N), (0, 0)))

    dst2 = dst.reshape(-1, C)

    h1 = _matmul(xp, W1, D_HID)
    pq1, gidx1 = _signs64(h1, src, dst)
    g1 = _gtable(h1, pq1, D_HID)
    op1 = _scatter64(g1, gidx1.reshape(-1, C), dst2)
    h2 = _combine_mm(op1, b1, W2, D_HID, D_OUT)
    pq2, gidx2 = _signs128(h2, src, dst)
    g2 = _gtable_split(h2, pq2)
    gidx2r = gidx2.reshape(-1, C)
    op2a = _scatter64(g2[0], gidx2r, dst2)
    op2b = _scatter64(g2[1], gidx2r, dst2)
    return _final(op2a, op2b, b2)


# speedup vs baseline: 2.6967x; 1.0141x over previous
import functools

import jax
import jax.numpy as jnp
from jax import lax
from jax.experimental import pallas as pl
from jax.experimental.pallas import tpu as pltpu
from jax.experimental.pallas import tpu_sc as plsc

N = 10000
E = 320000
D_IN = 128
D_HID = 64
D_OUT = 128
T = 1.0

NC = 2
NS = 16
NW = NC * NS
L = 16

NP = 10240
PAD = N
C = 128
E_TOT = E + N
CHUNKS = -(-E_TOT // (NW * C))
EP = NW * C * CHUNKS
CH0S = 107
CH1S = 2 * CHUNKS - CH0S
CHMS = max(CH0S, CH1S)
CMINS = min(CH0S, CH1S)
CH0 = 101
CH1 = 2 * CHUNKS - CH0
CHM = max(CH0, CH1)
CMIN = min(CH0, CH1)
EP_AL = EP + (max(CHMS, CHM) - min(CMINS, CMIN)) * C

_mesh = plsc.VectorSubcoreMesh(core_axis_name="c", subcore_axis_name="s")


CE = CHMS * C


def _signs_body(D, h_hbm, src_hbm, dst_hbm, pq_hbm, gidx_hbm,
                p_loc, q_loc, src_all, dst_all,
                srows0, srows1, drows0, drows1,
                sem10, sem11, sem20, sem21):
    c = lax.axis_index("c")
    s = lax.axis_index("s")
    w = c * NS + s
    my_ch = jnp.where(c == 0, CH0S, CH1S)
    base_chunk = jnp.where(c == 0, s * CH0S, NS * CH0S + s * CH1S)

    srowss = (srows0, srows1)
    drowss = (drows0, drows1)
    sem1s = (sem10, sem11)
    sem2s = (sem20, sem21)

    zero16 = jnp.zeros((L,), jnp.float32)

    def zloop(i, carry):
        p_loc[pl.ds(i * L, L)] = zero16
        q_loc[pl.ds(i * L, L)] = zero16
        return carry
    lax.fori_loop(0, NP // L, zloop, 0)

    e_base = base_chunk * C
    pltpu.sync_copy(src_hbm.at[pl.ds(e_base, CE)], src_all)
    pltpu.sync_copy(dst_hbm.at[pl.ds(e_base, CE)], dst_all)

    lane = lax.iota(jnp.int32, L)
    ones = jnp.ones((L,), jnp.float32)

    def issue(ci, b):
        pltpu.async_copy(
            h_hbm.at[src_all.at[pl.ds(ci * C, C)]], srowss[b], sem1s[b])
        pltpu.async_copy(
            h_hbm.at[dst_all.at[pl.ds(ci * C, C)]], drowss[b], sem2s[b])

    def compute(ci, b):
        srows = srowss[b]
        drows = drowss[b]

        def group(g, carry):
            dots = jnp.zeros((L,), jnp.float32)
            for l in range(L):
                e = g * L + l
                acc = (srows[e, pl.ds(0, L)] * drows[e, pl.ds(0, L)])
                for j in range(1, D // L):
                    acc = acc + (srows[e, pl.ds(j * L, L)]
                                 * drows[e, pl.ds(j * L, L)])
                dots = jnp.where(lane == l, jnp.sum(acc), dots)
            pos = dots > 0.0
            srcg = src_all[pl.ds(ci * C + g * L, L)]
            plsc.addupdate_scatter(p_loc, [srcg], ones, mask=pos)
            plsc.addupdate_scatter(q_loc, [srcg], ones,
                                   mask=jnp.logical_not(pos))
            src_all[pl.ds(ci * C + g * L, L)] = srcg + jnp.where(
                pos, jnp.int32(NP), jnp.int32(0))
            return carry
        lax.fori_loop(0, C // L, group, 0)

    issue(0, 0)

    def pair(pi, carry):
        for b in range(2):
            ci = pi * 2 + b

            @pl.when(ci < my_ch)
            def _():
                @pl.when(ci + 1 < my_ch)
                def _():
                    issue(ci + 1, b ^ 1)
                pltpu.make_async_copy(
                    h_hbm.at[src_all.at[pl.ds(0, C)]],
                    srowss[b], sem1s[b]).wait()
                pltpu.make_async_copy(
                    h_hbm.at[dst_all.at[pl.ds(0, C)]],
                    drowss[b], sem2s[b]).wait()
                compute(ci, b)
        return carry
    lax.fori_loop(0, (my_ch + 1) // 2, pair, 0)

    pltpu.sync_copy(src_all.at[pl.ds(0, CMINS * C)],
                    gidx_hbm.at[pl.ds(e_base, CMINS * C)])

    @pl.when(my_ch > CMINS)
    def _():
        pltpu.sync_copy(
            src_all.at[pl.ds(CMINS * C, (CHMS - CMINS) * C)],
            gidx_hbm.at[pl.ds(e_base + CMINS * C, (CHMS - CMINS) * C)])
    pltpu.sync_copy(p_loc, pq_hbm.at[w, 0])
    pltpu.sync_copy(q_loc, pq_hbm.at[w, 1])


def _make_signs(D):
    return pl.kernel(
        functools.partial(_signs_body, D),
        out_type=(
            jax.ShapeDtypeStruct((NW, 2, NP), jnp.float32),
            jax.ShapeDtypeStruct((EP_AL,), jnp.int32),
        ),
        mesh=_mesh,
        compiler_params=pltpu.CompilerParams(needs_layout_passes=False, use_tc_tiling_on_sc=False),
        scratch_types=[
            pltpu.VMEM((NP,), jnp.float32),
            pltpu.VMEM((NP,), jnp.float32),
            pltpu.VMEM((CE,), jnp.int32),
            pltpu.VMEM((CE,), jnp.int32),
            pltpu.VMEM((C, D), jnp.float32),
            pltpu.VMEM((C, D), jnp.float32),
            pltpu.VMEM((C, D), jnp.float32),
            pltpu.VMEM((C, D), jnp.float32),
            pltpu.SemaphoreType.DMA,
            pltpu.SemaphoreType.DMA,
            pltpu.SemaphoreType.DMA,
            pltpu.SemaphoreType.DMA,
        ],
    )


def _scatter_body(D, g_hbm, gidx2_hbm, dst2_hbm, out_hbm,
                  gidx_all, dst_all, rows0, rows1,
                  zbuf, osh, sem0, sem1):
    c = lax.axis_index("c")
    s = lax.axis_index("s")
    stripe = NP // NS
    my_ch = jnp.where(c == 0, CH0, CH1)
    base_chunk = jnp.where(c == 0, s * CH0, NS * CH0 + s * CH1)

    rowss = (rows0, rows1)
    sems = (sem0, sem1)

    zero16 = jnp.zeros((L,), jnp.float32)

    def zrow(r, carry):
        for j in range(D // L):
            zbuf[r, pl.ds(j * L, L)] = zero16
        return carry
    lax.fori_loop(0, stripe, zrow, 0)

    pltpu.sync_copy(zbuf, osh.at[pl.ds(s * stripe, stripe)])

    pltpu.sync_copy(gidx2_hbm.at[pl.ds(base_chunk, CHM)], gidx_all)
    pltpu.sync_copy(dst2_hbm.at[pl.ds(base_chunk, CHM)], dst_all)
    plsc.subcore_barrier()

    def issue(ci, b):
        pltpu.async_copy(g_hbm.at[gidx_all.at[ci]], rowss[b], sems[b])

    issue(0, 0)

    def pair(pi, carry):
        for b in range(2):
            ci = pi * 2 + b

            @pl.when(ci < my_ch)
            def _():
                @pl.when(ci + 1 < my_ch)
                def _():
                    issue(ci + 1, b ^ 1)
                pltpu.make_async_copy(
                    g_hbm.at[gidx_all.at[0]], rowss[b], sems[b]).wait()
                pltpu.sync_copy(rowss[b], osh.at[dst_all.at[ci]], add=True)
        return carry
    lax.fori_loop(0, (my_ch + 1) // 2, pair, 0)

    plsc.subcore_barrier()
    pltpu.sync_copy(osh.at[pl.ds(s * stripe, stripe)],
                    out_hbm.at[c, pl.ds(s * stripe, stripe)])


def _make_scatter(D):
    return pl.kernel(
        functools.partial(_scatter_body, D),
        out_type=jax.ShapeDtypeStruct((NC, NP, D), jnp.float32),
        mesh=_mesh,
        compiler_params=pltpu.CompilerParams(needs_layout_passes=False, use_tc_tiling_on_sc=False),
        scratch_types=[
            pltpu.VMEM((CHM, C), jnp.int32),
            pltpu.VMEM((CHM, C), jnp.int32),
            pltpu.VMEM((C, D), jnp.float32),
            pltpu.VMEM((C, D), jnp.float32),
            pltpu.VMEM((NP // NS, D), jnp.float32),
            pltpu.VMEM_SHARED((NP, D), jnp.float32),
            pltpu.SemaphoreType.DMA,
            pltpu.SemaphoreType.DMA,
        ],
    )


_BLK = 1024


def _mm_body(x_ref, w_ref, o_ref):
    o_ref[...] = lax.dot_general(
        x_ref[...], w_ref[...], (((1,), (1,)), ((), ())),
        preferred_element_type=jnp.float32)


def _matmul(x, w, d_out):
    npad, d_in = x.shape
    return pl.pallas_call(
        _mm_body,
        grid=(npad // _BLK,),
        in_specs=[
            pl.BlockSpec((_BLK, d_in), lambda i: (i, 0)),
            pl.BlockSpec((d_out, d_in), lambda i: (0, 0)),
        ],
        out_specs=pl.BlockSpec((_BLK, d_out), lambda i: (i, 0)),
        out_shape=jax.ShapeDtypeStruct((npad, d_out), jnp.float32),
    )(x, w)


def _gtable_body(h_ref, pq_ref, o_ref):
    g = pl.program_id(0)
    i = pl.program_id(1)
    p = jnp.sum(pq_ref[:, 0, pl.ds(i * _BLK, _BLK)], axis=0)
    q = jnp.sum(pq_ref[:, 1, pl.ds(i * _BLK, _BLK)], axis=0)
    amax = jnp.where(p > 0.0, T, -T)
    epos = jnp.exp(T - amax)
    eneg = jnp.exp(-T - amax)
    ssum = p * epos + q * eneg + 1e-16
    wsel = jnp.where(g == 0, eneg, epos) / ssum
    o_ref[...] = h_ref[...] * wsel[:, None]


def _gtable(h, pq, d):
    return pl.pallas_call(
        _gtable_body,
        grid=(2, NP // _BLK),
        in_specs=[
            pl.BlockSpec((_BLK, d), lambda g, i: (i, 0)),
            pl.BlockSpec((NW, 2, NP), lambda g, i: (0, 0, 0)),
        ],
        out_specs=pl.BlockSpec(
            (_BLK, d), lambda g, i: (g * (NP // _BLK) + i, 0)),
        out_shape=jax.ShapeDtypeStruct((2 * NP, d), jnp.float32),
    )(h, pq)


def _gtable_split_body(h_ref, pq_ref, o_ref):
    g = pl.program_id(0)
    hf = pl.program_id(1)
    i = pl.program_id(2)
    p = jnp.sum(pq_ref[:, 0, pl.ds(i * _BLK, _BLK)], axis=0)
    q = jnp.sum(pq_ref[:, 1, pl.ds(i * _BLK, _BLK)], axis=0)
    amax = jnp.where(p > 0.0, T, -T)
    epos = jnp.exp(T - amax)
    eneg = jnp.exp(-T - amax)
    ssum = p * epos + q * eneg + 1e-16
    wsel = jnp.where(g == 0, eneg, epos) / ssum
    half = D_OUT // 2
    hsel = jnp.where(hf == 0, h_ref[:, :half], h_ref[:, half:])
    o_ref[0] = hsel * wsel[:, None]


def _gtable_split(h, pq):
    return pl.pallas_call(
        _gtable_split_body,
        grid=(2, 2, NP // _BLK),
        in_specs=[
            pl.BlockSpec((_BLK, D_OUT), lambda g, hf, i: (i, 0)),
            pl.BlockSpec((NW, 2, NP), lambda g, hf, i: (0, 0, 0)),
        ],
        out_specs=pl.BlockSpec(
            (1, _BLK, D_OUT // 2),
            lambda g, hf, i: (hf, g * (NP // _BLK) + i, 0)),
        out_shape=jax.ShapeDtypeStruct((2, 2 * NP, D_OUT // 2), jnp.float32),
    )(h, pq)


def _combine_mm_body(op_ref, b_ref, w_ref, o_ref):
    i = pl.program_id(0)
    z = op_ref[0] + op_ref[1] + b_ref[...][None, :]
    z = jnp.maximum(z, 0.0)
    row = i * _BLK + lax.broadcasted_iota(jnp.int32, (_BLK, 1), 0)
    z = jnp.where(row < N, z, 0.0)
    o_ref[...] = lax.dot_general(
        z, w_ref[...], (((1,), (1,)), ((), ())),
        preferred_element_type=jnp.float32)


def _combine_mm(op, b, w, d_in, d_out):
    return pl.pallas_call(
        _combine_mm_body,
        grid=(NP // _BLK,),
        in_specs=[
            pl.BlockSpec((NC, _BLK, d_in), lambda i: (0, i, 0)),
            pl.BlockSpec((d_in,), lambda i: (0,)),
            pl.BlockSpec((d_out, d_in), lambda i: (0, 0)),
        ],
        out_specs=pl.BlockSpec((_BLK, d_out), lambda i: (i, 0)),
        out_shape=jax.ShapeDtypeStruct((NP, d_out), jnp.float32),
    )(op, b, w)


_OBLK = 400


def _final_body(opa_ref, opb_ref, b_ref, o_ref):
    z = jnp.concatenate(
        [opa_ref[0] + opa_ref[1], opb_ref[0] + opb_ref[1]], axis=1)
    z = z + b_ref[...][None, :]
    m = jnp.max(z, axis=1, keepdims=True)
    ez = jnp.exp(z - m)
    lse = jnp.log(jnp.sum(ez, axis=1, keepdims=True))
    o_ref[...] = z - m - lse


def _final(opa, opb, b):
    return pl.pallas_call(
        _final_body,
        grid=(N // _OBLK,),
        in_specs=[
            pl.BlockSpec((NC, _OBLK, D_OUT // 2), lambda i: (0, i, 0)),
            pl.BlockSpec((NC, _OBLK, D_OUT // 2), lambda i: (0, i, 0)),
            pl.BlockSpec((D_OUT,), lambda i: (0,)),
        ],
        out_specs=pl.BlockSpec((_OBLK, D_OUT), lambda i: (i, 0)),
        out_shape=jax.ShapeDtypeStruct((N, D_OUT), jnp.float32),
    )(opa, opb, b)


_signs64 = _make_signs(D_HID)
_signs128 = _make_signs(D_OUT)
_scatter64 = _make_scatter(D_HID)


def kernel(x, edge_index, W1, b1, W2, b2):
    loop = jnp.arange(N, dtype=jnp.int32)
    padv = jnp.full((EP_AL - E_TOT,), PAD, dtype=jnp.int32)
    src = jnp.concatenate([edge_index[0], loop, padv])
    dst = jnp.concatenate([edge_index[1], loop, padv])

    xp = jnp.pad(x, ((0, NP - N), (0, 0)))

    dst2 = dst.reshape(-1, C)

    h1 = _matmul(xp, W1, D_HID)
    pq1, gidx1 = _signs64(h1, src, dst)
    g1 = _gtable(h1, pq1, D_HID)
    op1 = _scatter64(g1, gidx1.reshape(-1, C), dst2)
    h2 = _combine_mm(op1, b1, W2, D_HID, D_OUT)
    pq2, gidx2 = _signs128(h2, src, dst)
    g2 = _gtable_split(h2, pq2)
    gidx2r = gidx2.reshape(-1, C)
    op2a = _scatter64(g2[0], gidx2r, dst2)
    op2b = _scatter64(g2[1], gidx2r, dst2)
    return _final(op2a, op2b, b2)


# speedup vs baseline: 2.7186x; 1.0081x over previous
import functools

import jax
import jax.numpy as jnp
from jax import lax
from jax.experimental import pallas as pl
from jax.experimental.pallas import tpu as pltpu
from jax.experimental.pallas import tpu_sc as plsc

N = 10000
E = 320000
D_IN = 128
D_HID = 64
D_OUT = 128
T = 1.0

NC = 2
NS = 16
NW = NC * NS
L = 16

NP = 10240
PAD = N
C = 128
E_TOT = E + N
CHUNKS = -(-E_TOT // (NW * C))
EP = NW * C * CHUNKS
CH0S = 109
CH1S = 2 * CHUNKS - CH0S
CHMS = max(CH0S, CH1S)
CMINS = min(CH0S, CH1S)
CH0 = 99
CH1 = 2 * CHUNKS - CH0
CHM = max(CH0, CH1)
CMIN = min(CH0, CH1)
EP_AL = EP + (max(CHMS, CHM) - min(CMINS, CMIN)) * C

_mesh = plsc.VectorSubcoreMesh(core_axis_name="c", subcore_axis_name="s")


CE = CHMS * C


def _signs_body(D, h_hbm, src_hbm, dst_hbm, pq_hbm, gidx_hbm,
                p_loc, q_loc, src_all, dst_all,
                srows0, srows1, drows0, drows1,
                sem10, sem11, sem20, sem21):
    c = lax.axis_index("c")
    s = lax.axis_index("s")
    w = c * NS + s
    my_ch = jnp.where(c == 0, CH0S, CH1S)
    base_chunk = jnp.where(c == 0, s * CH0S, NS * CH0S + s * CH1S)

    srowss = (srows0, srows1)
    drowss = (drows0, drows1)
    sem1s = (sem10, sem11)
    sem2s = (sem20, sem21)

    zero16 = jnp.zeros((L,), jnp.float32)

    def zloop(i, carry):
        p_loc[pl.ds(i * L, L)] = zero16
        q_loc[pl.ds(i * L, L)] = zero16
        return carry
    lax.fori_loop(0, NP // L, zloop, 0)

    e_base = base_chunk * C
    pltpu.sync_copy(src_hbm.at[pl.ds(e_base, CE)], src_all)
    pltpu.sync_copy(dst_hbm.at[pl.ds(e_base, CE)], dst_all)

    lane = lax.iota(jnp.int32, L)
    ones = jnp.ones((L,), jnp.float32)

    def issue(ci, b):
        pltpu.async_copy(
            h_hbm.at[src_all.at[pl.ds(ci * C, C)]], srowss[b], sem1s[b])
        pltpu.async_copy(
            h_hbm.at[dst_all.at[pl.ds(ci * C, C)]], drowss[b], sem2s[b])

    def compute(ci, b):
        srows = srowss[b]
        drows = drowss[b]

        def group(g, carry):
            dots = jnp.zeros((L,), jnp.float32)
            for l in range(L):
                e = g * L + l
                acc = (srows[e, pl.ds(0, L)] * drows[e, pl.ds(0, L)])
                for j in range(1, D // L):
                    acc = acc + (srows[e, pl.ds(j * L, L)]
                                 * drows[e, pl.ds(j * L, L)])
                dots = jnp.where(lane == l, jnp.sum(acc), dots)
            pos = dots > 0.0
            srcg = src_all[pl.ds(ci * C + g * L, L)]
            plsc.addupdate_scatter(p_loc, [srcg], ones, mask=pos)
            plsc.addupdate_scatter(q_loc, [srcg], ones,
                                   mask=jnp.logical_not(pos))
            src_all[pl.ds(ci * C + g * L, L)] = srcg + jnp.where(
                pos, jnp.int32(NP), jnp.int32(0))
            return carry
        lax.fori_loop(0, C // L, group, 0)

    issue(0, 0)

    def pair(pi, carry):
        for b in range(2):
            ci = pi * 2 + b

            @pl.when(ci < my_ch)
            def _():
                @pl.when(ci + 1 < my_ch)
                def _():
                    issue(ci + 1, b ^ 1)
                pltpu.make_async_copy(
                    h_hbm.at[src_all.at[pl.ds(0, C)]],
                    srowss[b], sem1s[b]).wait()
                pltpu.make_async_copy(
                    h_hbm.at[dst_all.at[pl.ds(0, C)]],
                    drowss[b], sem2s[b]).wait()
                compute(ci, b)
        return carry
    lax.fori_loop(0, (my_ch + 1) // 2, pair, 0)

    pltpu.sync_copy(src_all.at[pl.ds(0, CMINS * C)],
                    gidx_hbm.at[pl.ds(e_base, CMINS * C)])

    @pl.when(my_ch > CMINS)
    def _():
        pltpu.sync_copy(
            src_all.at[pl.ds(CMINS * C, (CHMS - CMINS) * C)],
            gidx_hbm.at[pl.ds(e_base + CMINS * C, (CHMS - CMINS) * C)])
    pltpu.sync_copy(p_loc, pq_hbm.at[w, 0])
    pltpu.sync_copy(q_loc, pq_hbm.at[w, 1])


def _make_signs(D):
    return pl.kernel(
        functools.partial(_signs_body, D),
        out_type=(
            jax.ShapeDtypeStruct((NW, 2, NP), jnp.float32),
            jax.ShapeDtypeStruct((EP_AL,), jnp.int32),
        ),
        mesh=_mesh,
        compiler_params=pltpu.CompilerParams(needs_layout_passes=False, use_tc_tiling_on_sc=False),
        scratch_types=[
            pltpu.VMEM((NP,), jnp.float32),
            pltpu.VMEM((NP,), jnp.float32),
            pltpu.VMEM((CE,), jnp.int32),
            pltpu.VMEM((CE,), jnp.int32),
            pltpu.VMEM((C, D), jnp.float32),
            pltpu.VMEM((C, D), jnp.float32),
            pltpu.VMEM((C, D), jnp.float32),
            pltpu.VMEM((C, D), jnp.float32),
            pltpu.SemaphoreType.DMA,
            pltpu.SemaphoreType.DMA,
            pltpu.SemaphoreType.DMA,
            pltpu.SemaphoreType.DMA,
        ],
    )


def _scatter_body(D, g_hbm, gidx2_hbm, dst2_hbm, out_hbm,
                  gidx_all, dst_all, rows0, rows1,
                  zbuf, osh, sem0, sem1):
    c = lax.axis_index("c")
    s = lax.axis_index("s")
    stripe = NP // NS
    my_ch = jnp.where(c == 0, CH0, CH1)
    base_chunk = jnp.where(c == 0, s * CH0, NS * CH0 + s * CH1)

    rowss = (rows0, rows1)
    sems = (sem0, sem1)

    zero16 = jnp.zeros((L,), jnp.float32)

    def zrow(r, carry):
        for j in range(D // L):
            zbuf[r, pl.ds(j * L, L)] = zero16
        return carry
    lax.fori_loop(0, stripe, zrow, 0)

    pltpu.sync_copy(zbuf, osh.at[pl.ds(s * stripe, stripe)])

    pltpu.sync_copy(gidx2_hbm.at[pl.ds(base_chunk, CHM)], gidx_all)
    pltpu.sync_copy(dst2_hbm.at[pl.ds(base_chunk, CHM)], dst_all)
    plsc.subcore_barrier()

    def issue(ci, b):
        pltpu.async_copy(g_hbm.at[gidx_all.at[ci]], rowss[b], sems[b])

    issue(0, 0)

    def pair(pi, carry):
        for b in range(2):
            ci = pi * 2 + b

            @pl.when(ci < my_ch)
            def _():
                @pl.when(ci + 1 < my_ch)
                def _():
                    issue(ci + 1, b ^ 1)
                pltpu.make_async_copy(
                    g_hbm.at[gidx_all.at[0]], rowss[b], sems[b]).wait()
                pltpu.sync_copy(rowss[b], osh.at[dst_all.at[ci]], add=True)
        return carry
    lax.fori_loop(0, (my_ch + 1) // 2, pair, 0)

    plsc.subcore_barrier()
    pltpu.sync_copy(osh.at[pl.ds(s * stripe, stripe)],
                    out_hbm.at[c, pl.ds(s * stripe, stripe)])


def _make_scatter(D):
    return pl.kernel(
        functools.partial(_scatter_body, D),
        out_type=jax.ShapeDtypeStruct((NC, NP, D), jnp.float32),
        mesh=_mesh,
        compiler_params=pltpu.CompilerParams(needs_layout_passes=False, use_tc_tiling_on_sc=False),
        scratch_types=[
            pltpu.VMEM((CHM, C), jnp.int32),
            pltpu.VMEM((CHM, C), jnp.int32),
            pltpu.VMEM((C, D), jnp.float32),
            pltpu.VMEM((C, D), jnp.float32),
            pltpu.VMEM((NP // NS, D), jnp.float32),
            pltpu.VMEM_SHARED((NP, D), jnp.float32),
            pltpu.SemaphoreType.DMA,
            pltpu.SemaphoreType.DMA,
        ],
    )


_BLK = 1024


def _mm_body(x_ref, w_ref, o_ref):
    o_ref[...] = lax.dot_general(
        x_ref[...], w_ref[...], (((1,), (1,)), ((), ())),
        preferred_element_type=jnp.float32)


def _matmul(x, w, d_out):
    npad, d_in = x.shape
    return pl.pallas_call(
        _mm_body,
        grid=(npad // _BLK,),
        in_specs=[
            pl.BlockSpec((_BLK, d_in), lambda i: (i, 0)),
            pl.BlockSpec((d_out, d_in), lambda i: (0, 0)),
        ],
        out_specs=pl.BlockSpec((_BLK, d_out), lambda i: (i, 0)),
        out_shape=jax.ShapeDtypeStruct((npad, d_out), jnp.float32),
    )(x, w)


def _gtable_body(h_ref, pq_ref, o_ref):
    g = pl.program_id(0)
    i = pl.program_id(1)
    p = jnp.sum(pq_ref[:, 0, pl.ds(i * _BLK, _BLK)], axis=0)
    q = jnp.sum(pq_ref[:, 1, pl.ds(i * _BLK, _BLK)], axis=0)
    amax = jnp.where(p > 0.0, T, -T)
    epos = jnp.exp(T - amax)
    eneg = jnp.exp(-T - amax)
    ssum = p * epos + q * eneg + 1e-16
    wsel = jnp.where(g == 0, eneg, epos) / ssum
    o_ref[...] = h_ref[...] * wsel[:, None]


def _gtable(h, pq, d):
    return pl.pallas_call(
        _gtable_body,
        grid=(2, NP // _BLK),
        in_specs=[
            pl.BlockSpec((_BLK, d), lambda g, i: (i, 0)),
            pl.BlockSpec((NW, 2, NP), lambda g, i: (0, 0, 0)),
        ],
        out_specs=pl.BlockSpec(
            (_BLK, d), lambda g, i: (g * (NP // _BLK) + i, 0)),
        out_shape=jax.ShapeDtypeStruct((2 * NP, d), jnp.float32),
    )(h, pq)


def _gtable_split_body(h_ref, pq_ref, o_ref):
    g = pl.program_id(0)
    hf = pl.program_id(1)
    i = pl.program_id(2)
    p = jnp.sum(pq_ref[:, 0, pl.ds(i * _BLK, _BLK)], axis=0)
    q = jnp.sum(pq_ref[:, 1, pl.ds(i * _BLK, _BLK)], axis=0)
    amax = jnp.where(p > 0.0, T, -T)
    epos = jnp.exp(T - amax)
    eneg = jnp.exp(-T - amax)
    ssum = p * epos + q * eneg + 1e-16
    wsel = jnp.where(g == 0, eneg, epos) / ssum
    half = D_OUT // 2
    hsel = jnp.where(hf == 0, h_ref[:, :half], h_ref[:, half:])
    o_ref[0] = hsel * wsel[:, None]


def _gtable_split(h, pq):
    return pl.pallas_call(
        _gtable_split_body,
        grid=(2, 2, NP // _BLK),
        in_specs=[
            pl.BlockSpec((_BLK, D_OUT), lambda g, hf, i: (i, 0)),
            pl.BlockSpec((NW, 2, NP), lambda g, hf, i: (0, 0, 0)),
        ],
        out_specs=pl.BlockSpec(
            (1, _BLK, D_OUT // 2),
            lambda g, hf, i: (hf, g * (NP // _BLK) + i, 0)),
        out_shape=jax.ShapeDtypeStruct((2, 2 * NP, D_OUT // 2), jnp.float32),
    )(h, pq)


def _combine_mm_body(op_ref, b_ref, w_ref, o_ref):
    i = pl.program_id(0)
    z = op_ref[0] + op_ref[1] + b_ref[...][None, :]
    z = jnp.maximum(z, 0.0)
    row = i * _BLK + lax.broadcasted_iota(jnp.int32, (_BLK, 1), 0)
    z = jnp.where(row < N, z, 0.0)
    o_ref[...] = lax.dot_general(
        z, w_ref[...], (((1,), (1,)), ((), ())),
        preferred_element_type=jnp.float32)


def _combine_mm(op, b, w, d_in, d_out):
    return pl.pallas_call(
        _combine_mm_body,
        grid=(NP // _BLK,),
        in_specs=[
            pl.BlockSpec((NC, _BLK, d_in), lambda i: (0, i, 0)),
            pl.BlockSpec((d_in,), lambda i: (0,)),
            pl.BlockSpec((d_out, d_in), lambda i: (0, 0)),
        ],
        out_specs=pl.BlockSpec((_BLK, d_out), lambda i: (i, 0)),
        out_shape=jax.ShapeDtypeStruct((NP, d_out), jnp.float32),
    )(op, b, w)


_OBLK = 400


def _final_body(opa_ref, opb_ref, b_ref, o_ref):
    z = jnp.concatenate(
        [opa_ref[0] + opa_ref[1], opb_ref[0] + opb_ref[1]], axis=1)
    z = z + b_ref[...][None, :]
    m = jnp.max(z, axis=1, keepdims=True)
    ez = jnp.exp(z - m)
    lse = jnp.log(jnp.sum(ez, axis=1, keepdims=True))
    o_ref[...] = z - m - lse


def _final(opa, opb, b):
    return pl.pallas_call(
        _final_body,
        grid=(N // _OBLK,),
        in_specs=[
            pl.BlockSpec((NC, _OBLK, D_OUT // 2), lambda i: (0, i, 0)),
            pl.BlockSpec((NC, _OBLK, D_OUT // 2), lambda i: (0, i, 0)),
            pl.BlockSpec((D_OUT,), lambda i: (0,)),
        ],
        out_specs=pl.BlockSpec((_OBLK, D_OUT), lambda i: (i, 0)),
        out_shape=jax.ShapeDtypeStruct((N, D_OUT), jnp.float32),
    )(opa, opb, b)


_signs64 = _make_signs(D_HID)
_signs128 = _make_signs(D_OUT)
_scatter64 = _make_scatter(D_HID)


def kernel(x, edge_index, W1, b1, W2, b2):
    loop = jnp.arange(N, dtype=jnp.int32)
    padv = jnp.full((EP_AL - E_TOT,), PAD, dtype=jnp.int32)
    src = jnp.concatenate([edge_index[0], loop, padv])
    dst = jnp.concatenate([edge_index[1], loop, padv])

    xp = jnp.pad(x, ((0, NP - N), (0, 0)))

    dst2 = dst.reshape(-1, C)

    h1 = _matmul(xp, W1, D_HID)
    pq1, gidx1 = _signs64(h1, src, dst)
    g1 = _gtable(h1, pq1, D_HID)
    op1 = _scatter64(g1, gidx1.reshape(-1, C), dst2)
    h2 = _combine_mm(op1, b1, W2, D_HID, D_OUT)
    pq2, gidx2 = _signs128(h2, src, dst)
    g2 = _gtable_split(h2, pq2)
    gidx2r = gidx2.reshape(-1, C)
    op2a = _scatter64(g2[0], gidx2r, dst2)
    op2b = _scatter64(g2[1], gidx2r, dst2)
    return _final(op2a, op2b, b2)
